# baseline (speedup 1.0000x reference)
import functools

import jax
import jax.numpy as jnp
from jax import lax
from jax.experimental import pallas as pl
from jax.experimental.pallas import tpu as pltpu
from jax.experimental.pallas import tpu_sc as plsc

_V = 1000000
_D = 32
_BATCH = 16384
_HIST = 50
_NC = 2
_NS = 16
_NW = _NC * _NS

_NT = 7813
_TPW = 245
_LROWS = _NT * 32

_GG = 7
_NBC = _BATCH // 128
_UPW = (_GG * _NBC) // _NW

_mesh = plsc.VectorSubcoreMesh(core_axis_name="c", subcore_axis_name="s")


def _wid():
    return lax.axis_index("s") * _NC + lax.axis_index("c")


@functools.partial(
    pl.kernel,
    mesh=_mesh,
    out_type=jax.ShapeDtypeStruct((_LROWS, 128), jnp.float32),
    scratch_types=[
        [pltpu.VMEM((_D, 128), jnp.float32)] * 2,
        [pltpu.VMEM((_D, 128), jnp.float32)] * 2,
        [pltpu.SemaphoreType.DMA] * 2,
        [pltpu.SemaphoreType.DMA] * 2,
    ],
    compiler_params=pltpu.CompilerParams(needs_layout_passes=False),
)
def _relayout(table_hbm, lin_hbm, src_v, dst_v, sem_i, sem_o):
    w = _wid()
    base = w * _TPW
    nv = jnp.minimum(_TPW, _NT - base)

    iota = lax.iota(jnp.int32, 16)
    rowk = [iota + 16 * (k & 1) for k in range(8)]
    uk = [jnp.full((16,), k >> 1, jnp.int32) for k in range(8)]

    def fire_in(c, b):
        off = pl.multiple_of(c * 128, 128)
        pltpu.async_copy(table_hbm.at[:, pl.ds(off, 128)], src_v[b], sem_i[b])

    def wait_in(b):
        pltpu.make_async_copy(
            table_hbm.at[:, pl.ds(0, 128)], src_v[b], sem_i[b]).wait()

    def fire_out(c, b):
        off = pl.multiple_of(c * 32, 32)
        pltpu.async_copy(dst_v[b], lin_hbm.at[pl.ds(off, 32)], sem_o[b])

    def wait_out(b):
        pltpu.make_async_copy(
            dst_v[b], lin_hbm.at[pl.ds(0, 32)], sem_o[b]).wait()

    fire_in(base, 0)

    def transpose_chunk(b):
        @plsc.parallel_loop(0, _D, unroll=8)
        def per_q(q):
            q4 = jnp.full((16,), 4 * q, jnp.int32)
            vals = [
                plsc.load_gather(src_v[b], [rowk[k], q4 + uk[k]])
                for k in range(8)
            ]
            for k in range(8):
                dst_v[b][q, pl.ds(16 * k, 16)] = vals[k]

    def pair(i2, carry):
        for b in range(2):
            i = i2 * 2 + b
            c = base + i

            @pl.when(i + 1 < nv)
            def _():
                fire_in(c + 1, 1 - b)

            @pl.when(i < nv)
            def _():
                wait_in(b)

                @pl.when(i >= 2)
                def _():
                    wait_out(b)

                transpose_chunk(b)
                fire_out(c, b)
        return carry

    lax.fori_loop(0, (_TPW + 1) // 2, pair, 0)
    wait_out(0)
    wait_out(1)


@functools.partial(
    pl.kernel,
    mesh=_mesh,
    out_type=jax.ShapeDtypeStruct((_HIST, _D, _BATCH), jnp.float32),
    scratch_types=[
        pltpu.VMEM((8, 128), jnp.int32),
        pltpu.VMEM((8, 128), jnp.int32),
        pltpu.VMEM((8, 128), jnp.int32),
        [pltpu.VMEM((128, 128), jnp.float32)] * 2,
        [pltpu.VMEM((_D, 128), jnp.float32)] * 2,
        pltpu.SemaphoreType.DMA,
        [pltpu.SemaphoreType.DMA] * 2,
        [pltpu.SemaphoreType.DMA] * 2,
    ],
    compiler_params=pltpu.CompilerParams(needs_layout_passes=False),
)
def _gather(lin_hbm, idx_hbm, out_hbm, idx_v, gidx_v, rem_v, gath_v, tr_v,
            sem_x, sem_g, sem_o):
    w = _wid()
    iota = lax.iota(jnp.int32, 16)
    rowk = [iota + 16 * k for k in range(8)]

    def fire_gather(s, b):
        pltpu.async_copy(lin_hbm.at[gidx_v.at[s]], gath_v[b], sem_g[b])

    def wait_gather(b):
        pltpu.make_async_copy(
            lin_hbm.at[gidx_v.at[0]], gath_v[b], sem_g[b]).wait()

    def fire_out(h, bc, b):
        off = pl.multiple_of(bc * 128, 128)
        pltpu.async_copy(tr_v[b], out_hbm.at[h, :, pl.ds(off, 128)], sem_o[b])

    def wait_out(b):
        pltpu.make_async_copy(
            tr_v[b], out_hbm.at[0, :, pl.ds(0, 128)], sem_o[b]).wait()

    def unit(u_i, carry):
        u = u_i * _NW + w
        g = u // _NBC
        bc = u % _NBC
        goff = pl.multiple_of(g * 8, 8)
        boff = pl.multiple_of(bc * 128, 128)
        pltpu.async_copy(
            idx_hbm.at[pl.ds(goff, 8), pl.ds(boff, 128)], idx_v, sem_x).wait()

        for s8 in range(8):
            for k in range(8):
                v = idx_v[s8, pl.ds(16 * k, 16)]
                gidx_v[s8, pl.ds(16 * k, 16)] = v >> 2
                rem_v[s8, pl.ds(16 * k, 16)] = (v & 3) * 32

        def valid(s):
            return g * 8 + s < _HIST

        @pl.when(valid(0))
        def _():
            fire_gather(0, 0)

        for s in range(8):
            b = s % 2
            b2 = b
            if s + 1 < 8:
                @pl.when(valid(s + 1))
                def _():
                    fire_gather(s + 1, 1 - b)

            @pl.when(valid(s))
            def _():
                wait_gather(b)
                if s >= 2:
                    wait_out(b2)

                rv = [rem_v[s, pl.ds(16 * k, 16)] for k in range(8)]

                @plsc.parallel_loop(0, _D, unroll=8)
                def per_d(d):
                    vals = [
                        plsc.load_gather(gath_v[b], [rowk[k], rv[k] + d])
                        for k in range(8)
                    ]
                    for k in range(8):
                        tr_v[b2][d, pl.ds(16 * k, 16)] = vals[k]
                fire_out(g * 8 + s, bc, b2)

        wait_out(0)
        wait_out(1)
        return carry

    lax.fori_loop(0, _UPW, unit, 0)


def kernel(inputs, embedding_matrix):
    table_t = embedding_matrix.T
    lin = _relayout(table_t)
    raw = _gather(lin, inputs.T)
    return jnp.transpose(raw, (2, 0, 1))


# speedup vs baseline: 1.0246x; 1.0246x over previous
import functools

import jax
import jax.numpy as jnp
from jax import lax
from jax.experimental import pallas as pl
from jax.experimental.pallas import tpu as pltpu
from jax.experimental.pallas import tpu_sc as plsc

_V = 1000000
_D = 32
_BATCH = 16384
_HIST = 50
_NC = 2
_NS = 16
_NW = _NC * _NS

_NT = 7813
_TPW = 245
_LROWS = _NT * 32

_GG = 7
_NBC = _BATCH // 128
_UPW = (_GG * _NBC) // _NW

_mesh = plsc.VectorSubcoreMesh(core_axis_name="c", subcore_axis_name="s")


def _wid():
    return lax.axis_index("s") * _NC + lax.axis_index("c")


@functools.partial(
    pl.kernel,
    mesh=_mesh,
    out_type=jax.ShapeDtypeStruct((_LROWS, 128), jnp.float32),
    scratch_types=[
        [pltpu.VMEM((_D, 128), jnp.float32)] * 2,
        [pltpu.VMEM((_D, 128), jnp.float32)] * 2,
        [pltpu.SemaphoreType.DMA] * 2,
        [pltpu.SemaphoreType.DMA] * 2,
    ],
    compiler_params=pltpu.CompilerParams(needs_layout_passes=False),
)
def _relayout(table_hbm, lin_hbm, src_v, dst_v, sem_i, sem_o):
    w = _wid()
    base = w * _TPW
    nv = jnp.minimum(_TPW, _NT - base)

    iota = lax.iota(jnp.int32, 16)
    rowk = [iota + 16 * (k & 1) for k in range(8)]
    uk = [jnp.full((16,), k >> 1, jnp.int32) for k in range(8)]

    def fire_in(c, b):
        off = pl.multiple_of(c * 128, 128)
        pltpu.async_copy(table_hbm.at[:, pl.ds(off, 128)], src_v[b], sem_i[b])

    def wait_in(b):
        pltpu.make_async_copy(
            table_hbm.at[:, pl.ds(0, 128)], src_v[b], sem_i[b]).wait()

    def fire_out(c, b):
        off = pl.multiple_of(c * 32, 32)
        pltpu.async_copy(dst_v[b], lin_hbm.at[pl.ds(off, 32)], sem_o[b])

    def wait_out(b):
        pltpu.make_async_copy(
            dst_v[b], lin_hbm.at[pl.ds(0, 32)], sem_o[b]).wait()

    fire_in(base, 0)

    def transpose_chunk(b):
        @plsc.parallel_loop(0, _D, unroll=4)
        def per_q(q):
            q4 = jnp.full((16,), 4 * q, jnp.int32)
            vals = [
                plsc.load_gather(src_v[b], [rowk[k], q4 + uk[k]])
                for k in range(8)
            ]
            for k in range(8):
                dst_v[b][q, pl.ds(16 * k, 16)] = vals[k]

    def pair(i2, carry):
        for b in range(2):
            i = i2 * 2 + b
            c = base + i

            @pl.when(i + 1 < nv)
            def _():
                fire_in(c + 1, 1 - b)

            @pl.when(i < nv)
            def _():
                wait_in(b)

                @pl.when(i >= 2)
                def _():
                    wait_out(b)

                transpose_chunk(b)
                fire_out(c, b)
        return carry

    lax.fori_loop(0, (_TPW + 1) // 2, pair, 0)
    wait_out(0)
    wait_out(1)


@functools.partial(
    pl.kernel,
    mesh=_mesh,
    out_type=jax.ShapeDtypeStruct((_HIST, _D, _BATCH), jnp.float32),
    scratch_types=[
        pltpu.VMEM((8, 128), jnp.int32),
        pltpu.VMEM((8, 128), jnp.int32),
        pltpu.VMEM((8, 128), jnp.int32),
        [pltpu.VMEM((128, 128), jnp.float32)] * 2,
        [pltpu.VMEM((_D, 128), jnp.float32)] * 2,
        pltpu.SemaphoreType.DMA,
        [pltpu.SemaphoreType.DMA] * 2,
        [pltpu.SemaphoreType.DMA] * 2,
    ],
    compiler_params=pltpu.CompilerParams(needs_layout_passes=False),
)
def _gather(lin_hbm, idx_hbm, out_hbm, idx_v, gidx_v, rem_v, gath_v, tr_v,
            sem_x, sem_g, sem_o):
    w = _wid()
    iota = lax.iota(jnp.int32, 16)
    rowk = [iota + 16 * k for k in range(8)]

    def fire_gather(s, b):
        pltpu.async_copy(lin_hbm.at[gidx_v.at[s]], gath_v[b], sem_g[b])

    def wait_gather(b):
        pltpu.make_async_copy(
            lin_hbm.at[gidx_v.at[0]], gath_v[b], sem_g[b]).wait()

    def fire_out(h, bc, b):
        off = pl.multiple_of(bc * 128, 128)
        pltpu.async_copy(tr_v[b], out_hbm.at[h, :, pl.ds(off, 128)], sem_o[b])

    def wait_out(b):
        pltpu.make_async_copy(
            tr_v[b], out_hbm.at[0, :, pl.ds(0, 128)], sem_o[b]).wait()

    def unit(u_i, carry):
        u = u_i * _NW + w
        g = u // _NBC
        bc = u % _NBC
        goff = pl.multiple_of(g * 8, 8)
        boff = pl.multiple_of(bc * 128, 128)
        pltpu.async_copy(
            idx_hbm.at[pl.ds(goff, 8), pl.ds(boff, 128)], idx_v, sem_x).wait()

        for s8 in range(8):
            for k in range(8):
                v = idx_v[s8, pl.ds(16 * k, 16)]
                gidx_v[s8, pl.ds(16 * k, 16)] = v >> 2
                rem_v[s8, pl.ds(16 * k, 16)] = (v & 3) * 32

        def valid(s):
            return g * 8 + s < _HIST

        @pl.when(valid(0))
        def _():
            fire_gather(0, 0)

        for s in range(8):
            b = s % 2
            b2 = b
            if s + 1 < 8:
                @pl.when(valid(s + 1))
                def _():
                    fire_gather(s + 1, 1 - b)

            @pl.when(valid(s))
            def _():
                wait_gather(b)
                if s >= 2:
                    wait_out(b2)

                rv = [rem_v[s, pl.ds(16 * k, 16)] for k in range(8)]

                @plsc.parallel_loop(0, _D, unroll=4)
                def per_d(d):
                    vals = [
                        plsc.load_gather(gath_v[b], [rowk[k], rv[k] + d])
                        for k in range(8)
                    ]
                    for k in range(8):
                        tr_v[b2][d, pl.ds(16 * k, 16)] = vals[k]
                fire_out(g * 8 + s, bc, b2)

        wait_out(0)
        wait_out(1)
        return carry

    lax.fori_loop(0, _UPW, unit, 0)


def kernel(inputs, embedding_matrix):
    table_t = embedding_matrix.T
    lin = _relayout(table_t)
    raw = _gather(lin, inputs.T)
    return jnp.transpose(raw, (2, 0, 1))
